# baseline (speedup 1.0000x reference)
import functools

import jax
import jax.numpy as jnp
from jax import lax
from jax.experimental import pallas as pl
from jax.experimental.pallas import tpu as pltpu, tpu_sc as plsc

_EMBED = 64
_L = 200
_HALF = 100
_NVEC = _EMBED // 16


def _sc_body(seq_hbm, table_hbm, out_hbm, idx_a, idx_b, rows_a, rows_b,
             blk_v, sem):
    nc = 2
    wid = lax.axis_index("s") * nc + lax.axis_index("c")
    n_rows = blk_v.shape[0]
    base = wid * n_rows

    def seq_step(s_local, _):
        row = base + s_local
        pltpu.sync_copy(seq_hbm.at[row, 0], idx_a)
        pltpu.sync_copy(seq_hbm.at[row, 1], idx_b)
        cp_a = pltpu.async_copy(table_hbm.at[idx_a], rows_a, sem)
        cp_b = pltpu.async_copy(table_hbm.at[idx_b], rows_b, sem)
        cp_a.wait()
        cp_b.wait()
        accs = tuple(jnp.zeros((16,), jnp.float32) for _ in range(_NVEC))
        for rows in (rows_a, rows_b):
            def inner(r, acc, rows=rows):
                return tuple(
                    a + rows[r, pl.ds(16 * k, 16)]
                    for k, a in enumerate(acc))
            accs = lax.fori_loop(0, _HALF, inner, accs)
        for k in range(_NVEC):
            blk_v[s_local, pl.ds(16 * k, 16)] = accs[k]
        return 0

    lax.fori_loop(0, n_rows, seq_step, 0)
    pltpu.sync_copy(blk_v, out_hbm.at[pl.ds(base, n_rows)])


def _sc_sum(seq3, emb_table):
    batch, _, _ = seq3.shape
    n_workers = 32
    rows_per_w = batch // n_workers
    mesh = plsc.VectorSubcoreMesh(core_axis_name="c", subcore_axis_name="s")
    grid_kernel = functools.partial(
        pl.kernel,
        out_type=jax.ShapeDtypeStruct((batch, _EMBED), jnp.float32),
        mesh=mesh,
        scratch_types=[
            pltpu.VMEM((_HALF,), jnp.int32),
            pltpu.VMEM((_HALF,), jnp.int32),
            pltpu.VMEM((_HALF, _EMBED), jnp.float32),
            pltpu.VMEM((_HALF, _EMBED), jnp.float32),
            pltpu.VMEM((rows_per_w, _EMBED), jnp.float32),
            pltpu.SemaphoreType.DMA,
        ],
        compiler_params=pltpu.CompilerParams(use_tc_tiling_on_sc=False),
    )
    return grid_kernel(_sc_body)(seq3, emb_table)


def _tc_body(sum_ref, seq_ref, w_ref, b_ref, out_ref):
    valid = (seq_ref[...] != 0).astype(jnp.float32)
    counts = jnp.maximum(jnp.sum(valid, axis=1, keepdims=True), 1.0)
    enc = sum_ref[...] / counts
    logits = lax.dot_general(enc, w_ref[...], (((1,), (1,)), ((), ())),
                             preferred_element_type=jnp.float32)
    out_ref[...] = logits + b_ref[...]


def _tc_classify(summed, sequences, W, b):
    batch = summed.shape[0]
    n_classes = W.shape[0]
    return pl.pallas_call(
        _tc_body,
        out_shape=jax.ShapeDtypeStruct((batch, n_classes), jnp.float32),
    )(summed, sequences, W, b.reshape(1, n_classes))


@jax.jit
def kernel(sequences, emb_table, W, b):
    batch, seq_len = sequences.shape
    seq3 = sequences.reshape(batch, seq_len // _HALF, _HALF)
    summed = _sc_sum(seq3, emb_table)
    return _tc_classify(summed, sequences, W, b)


# speedup vs baseline: 1.3149x; 1.3149x over previous
import functools

import jax
import jax.numpy as jnp
from jax import lax
from jax.experimental import pallas as pl
from jax.experimental.pallas import tpu as pltpu, tpu_sc as plsc

_EMBED = 64
_L = 200
_HALF = 100
_NVEC = _EMBED // 16


_UNROLL = 4


def _sc_body(seq_hbm, table_hbm, out_hbm, idx_blk, rows0, rows1,
             blk_v, sem0, sem1):
    nc = 2
    wid = lax.axis_index("s") * nc + lax.axis_index("c")
    n_rows = blk_v.shape[0]
    base = wid * n_rows

    pltpu.sync_copy(seq_hbm.at[pl.ds(base, n_rows)], idx_blk)

    rows_bufs = (rows0, rows1)
    sems = (sem0, sem1)

    def fire(s, par):
        for j in range(2):
            pltpu.async_copy(table_hbm.at[idx_blk.at[s, j]],
                             rows_bufs[par].at[j], sems[par])

    def wait_buf(par):
        for j in range(2):
            pltpu.make_async_copy(table_hbm.at[pl.ds(0, _HALF)],
                                  rows_bufs[par].at[j], sems[par]).wait()

    fire(0, 0)
    fire(1, 1)

    def pair_step(p, _):
        for par in range(2):
            s = 2 * p + par
            wait_buf(par)
            rows = rows_bufs[par]
            accs = tuple(jnp.zeros((16,), jnp.float32) for _ in range(_NVEC))
            for j in range(2):
                def inner(r4, acc, rows=rows, j=j):
                    for dr in range(_UNROLL):
                        acc = tuple(
                            a + rows[j, r4 * _UNROLL + dr, pl.ds(16 * k, 16)]
                            for k, a in enumerate(acc))
                    return acc
                accs = lax.fori_loop(0, _HALF // _UNROLL, inner, accs)
            for k in range(_NVEC):
                blk_v[s, pl.ds(16 * k, 16)] = accs[k]

            @pl.when(s + 2 < n_rows)
            def _():
                fire(s + 2, par)
        return 0

    lax.fori_loop(0, n_rows // 2, pair_step, 0)
    pltpu.sync_copy(blk_v, out_hbm.at[pl.ds(base, n_rows)])


def _sc_sum(seq3, emb_table):
    batch, _, _ = seq3.shape
    n_workers = 32
    rows_per_w = batch // n_workers
    mesh = plsc.VectorSubcoreMesh(core_axis_name="c", subcore_axis_name="s")
    grid_kernel = functools.partial(
        pl.kernel,
        out_type=jax.ShapeDtypeStruct((batch, _EMBED), jnp.float32),
        mesh=mesh,
        scratch_types=[
            pltpu.VMEM((rows_per_w, 2, _HALF), jnp.int32),
            pltpu.VMEM((2, _HALF, _EMBED), jnp.float32),
            pltpu.VMEM((2, _HALF, _EMBED), jnp.float32),
            pltpu.VMEM((rows_per_w, _EMBED), jnp.float32),
            pltpu.SemaphoreType.DMA,
            pltpu.SemaphoreType.DMA,
        ],
        compiler_params=pltpu.CompilerParams(use_tc_tiling_on_sc=False),
    )
    return grid_kernel(_sc_body)(seq3, emb_table)


def _tc_body(sum_ref, seq_ref, w_ref, b_ref, out_ref):
    valid = (seq_ref[...] != 0).astype(jnp.float32)
    counts = jnp.maximum(jnp.sum(valid, axis=1, keepdims=True), 1.0)
    enc = sum_ref[...] / counts
    logits = lax.dot_general(enc, w_ref[...], (((1,), (1,)), ((), ())),
                             preferred_element_type=jnp.float32)
    out_ref[...] = logits + b_ref[...]


def _tc_classify(summed, sequences, W, b):
    batch = summed.shape[0]
    n_classes = W.shape[0]
    return pl.pallas_call(
        _tc_body,
        out_shape=jax.ShapeDtypeStruct((batch, n_classes), jnp.float32),
    )(summed, sequences, W, b.reshape(1, n_classes))


@jax.jit
def kernel(sequences, emb_table, W, b):
    batch, seq_len = sequences.shape
    seq3 = sequences.reshape(batch, seq_len // _HALF, _HALF)
    summed = _sc_sum(seq3, emb_table)
    return _tc_classify(summed, sequences, W, b)
